# baseline (speedup 1.0000x reference)
import jax
import jax.numpy as jnp
from jax import lax
from jax.experimental import pallas as pl
from jax.experimental.pallas import tpu as pltpu

_B = 32
_N = 4096
_GH = 512
_GW = 512
_POOL = 8
_SLAB = 4
_STEPS = _B // _SLAB


def _select_slab(slab, num_ref, wgt_ref, conv_ref, tran_ref, pooled_ref,
                 main_ref, fb_ref):
    conv = conv_ref[pl.ds(slab * _SLAB, _SLAB)]
    tran = tran_ref[pl.ds(slab * _SLAB, _SLAB)]
    gt = pooled_ref[pl.ds(slab * _SLAB, _SLAB)]
    k = jnp.maximum(num_ref[0], 1)
    w = wgt_ref[0]

    e1 = jnp.abs(gt - conv)
    e2 = jnp.abs(gt - tran)
    bits1 = lax.bitcast_convert_type(e1, jnp.int32)
    bits2 = lax.bitcast_convert_type(e2, jnp.int32)

    def step(i, carry):
        p1, p2 = carry
        bit = jnp.left_shift(jnp.int32(1), 30 - i)
        c1 = p1 | bit
        cnt1 = jnp.sum((bits1 >= c1).astype(jnp.int32), axis=(1, 2),
                       keepdims=True)
        p1 = jnp.where(cnt1 >= k, c1, p1)
        c2 = p2 | bit
        cnt2 = jnp.sum((bits2 >= c2).astype(jnp.int32), axis=(1, 2),
                       keepdims=True)
        p2 = jnp.where(cnt2 >= k, c2, p2)
        return (p1, p2)

    zero = jnp.zeros((_SLAB, 1, 1), jnp.int32)
    t1, t2 = lax.fori_loop(0, 31, step, (zero, zero))

    mask1 = bits1 >= t1
    mask2 = bits2 >= t2
    comb_tran = w * tran + (1.0 - w) * gt
    comb_conv = w * conv + (1.0 - w) * gt
    d_cg = conv - gt
    d_tg = tran - gt
    fb_ref[0] += jnp.sum(d_cg * d_cg) + jnp.sum(d_tg * d_tg)
    m1 = jnp.where(mask1, conv - comb_tran, d_cg)
    m2 = jnp.where(mask2, tran - comb_conv, d_tg)
    main_ref[0] += jnp.sum(m1 * m1) + jnp.sum(m2 * m2)


def _body(num_ref, wgt_ref, conv_ref, tran_ref, gt_ref, main_ref, fb_ref,
          pooled_ref):
    b = pl.program_id(0)

    @pl.when(b == 0)
    def _():
        main_ref[0] = 0.0
        fb_ref[0] = 0.0

    g2 = jnp.reshape(gt_ref[...], (_SLAB * _GH, _GW))
    ri = lax.broadcasted_iota(jnp.int32, (_GW, 64), 0)
    ci = lax.broadcasted_iota(jnp.int32, (_GW, 64), 1)
    pmat = jnp.where(ri // _POOL == ci, 1.0, 0.0).astype(jnp.float32)
    a = lax.dot_general(g2, pmat, (((1,), (0,)), ((), ())))
    s = jnp.sum(jnp.reshape(a, (256, _POOL, 64)), axis=1)
    s2 = jnp.reshape(s, (128, 2, 64))
    qq = jnp.concatenate([s2[:, 0, :], s2[:, 1, :]], axis=1)
    pooled_ref[pl.ds(b * _SLAB, _SLAB)] = jnp.reshape(qq, (_SLAB, 32, 128))

    @pl.when(b > 0)
    def _():
        _select_slab(b - 1, num_ref, wgt_ref, conv_ref, tran_ref,
                     pooled_ref, main_ref, fb_ref)

    @pl.when(b == _STEPS - 1)
    def _():
        _select_slab(_STEPS - 1, num_ref, wgt_ref, conv_ref, tran_ref,
                     pooled_ref, main_ref, fb_ref)


def kernel(dmap_conv, dmap_tran, gt_density, process):
    conv = dmap_conv.reshape(_B, 32, 128)
    tran = dmap_tran.reshape(_B, 32, 128)
    gt = gt_density.reshape(_B, _GH, _GW)
    p = process.astype(jnp.float32)
    num = jnp.floor(_N * (0.1 * p)).astype(jnp.int32)
    wgt = 1.0 * p

    main, fb = pl.pallas_call(
        _body,
        grid=(_STEPS,),
        in_specs=[
            pl.BlockSpec(memory_space=pltpu.SMEM),
            pl.BlockSpec(memory_space=pltpu.SMEM),
            pl.BlockSpec((_B, 32, 128), lambda b: (0, 0, 0)),
            pl.BlockSpec((_B, 32, 128), lambda b: (0, 0, 0)),
            pl.BlockSpec((_SLAB, _GH, _GW), lambda b: (b, 0, 0)),
        ],
        out_specs=[
            pl.BlockSpec(memory_space=pltpu.SMEM),
            pl.BlockSpec(memory_space=pltpu.SMEM),
        ],
        out_shape=[
            jax.ShapeDtypeStruct((1,), jnp.float32),
            jax.ShapeDtypeStruct((1,), jnp.float32),
        ],
        scratch_shapes=[pltpu.VMEM((_B, 32, 128), jnp.float32)],
        compiler_params=pltpu.CompilerParams(
            dimension_semantics=("arbitrary",)),
    )(num, wgt, conv, tran, gt)

    return jnp.where(num[0] < 1, fb[0], main[0])
